# Initial kernel scaffold; baseline (speedup 1.0000x reference)
import jax
import jax.numpy as jnp
from jax import lax
from jax.experimental import pallas as pl
from jax.experimental.pallas import tpu as pltpu
from jax.experimental.pallas import tpu_sc as plsc

_N = 50000
_E = 800000
_C = 256
_FA = 64
_FN = 16
_H2 = 128
_EPS = 1e-5
_f32 = jnp.float32
_i32 = jnp.int32


def _softplus(x):
    return jnp.maximum(x, 0.0) + jnp.log(1.0 + jnp.exp(-jnp.abs(x)))


def _sigmoid(x):
    return 1.0 / (1.0 + jnp.exp(-x))


def _rows(block_rows, width):
    return pl.BlockSpec((block_rows, width), lambda i: (i, 0))


def _const(shape):
    return pl.BlockSpec(shape, lambda i: tuple(0 for _ in shape))


def _embed_body(a_ref, w_ref, b_ref, o_ref):
    o_ref[...] = jnp.dot(a_ref[...], w_ref[...],
                         preferred_element_type=_f32) + b_ref[...]


def _embed(atom_fea, w, b2):
    return pl.pallas_call(
        _embed_body,
        grid=(125,),
        in_specs=[_rows(400, 128), _const((128, 64)), _const((1, 64))],
        out_specs=_rows(400, 64),
        out_shape=jax.ShapeDtypeStruct((_N, _FA), _f32),
    )(atom_fea, w, b2)


def _atom_mm_body(x_ref, ws_ref, wn_ref, xs_ref, xn_ref):
    x = x_ref[...]
    xs_ref[...] = jnp.dot(x, ws_ref[...], preferred_element_type=_f32)
    xn_ref[...] = jnp.dot(x, wn_ref[...], preferred_element_type=_f32)


def _atom_mm(x, ws, wn):
    return pl.pallas_call(
        _atom_mm_body,
        grid=(125,),
        in_specs=[_rows(400, _FA), _const((_FA, _H2)), _const((_FA, _H2))],
        out_specs=[_rows(400, _H2), _rows(400, _H2)],
        out_shape=[jax.ShapeDtypeStruct((_N, _H2), _f32),
                   jax.ShapeDtypeStruct((_N, _H2), _f32)],
    )(x, ws, wn)


def _stats_body(ep_ref, nf_ref, wf_ref, b_ref, o_ref):
    e = ep_ref[...] + jnp.dot(nf_ref[...], wf_ref[...],
                              preferred_element_type=_f32) + b_ref[...]
    s = jnp.concatenate([jnp.sum(e, axis=0, keepdims=True),
                         jnp.sum(e * e, axis=0, keepdims=True)], axis=0)

    @pl.when(pl.program_id(0) == 0)
    def _():
        o_ref[...] = s

    @pl.when(pl.program_id(0) > 0)
    def _():
        o_ref[...] += s


def _stats(ep, nf, wf, b2):
    return pl.pallas_call(
        _stats_body,
        grid=(400,),
        in_specs=[_rows(2000, _H2), _rows(2000, _FN),
                  _const((_FN, _H2)), _const((1, _H2))],
        out_specs=_const((2, _H2)),
        out_shape=jax.ShapeDtypeStruct((2, _H2), _f32),
    )(ep, nf, wf, b2)


def _apply_body(ep_ref, nf_ref, wf_ref, b_ref, st_ref, g1_ref, b1_ref, o_ref):
    e = ep_ref[...] + jnp.dot(nf_ref[...], wf_ref[...],
                              preferred_element_type=_f32) + b_ref[...]
    mean = st_ref[0:1, :] * (1.0 / _E)
    var = st_ref[1:2, :] * (1.0 / _E) - mean * mean
    ebn = (e - mean) * (lax.rsqrt(var + _EPS) * g1_ref[...]) + b1_ref[...]
    o_ref[...] = _sigmoid(ebn[:, :_FA]) * _softplus(ebn[:, _FA:])


def _apply(ep, nf, wf, b2, st, g1, b1):
    return pl.pallas_call(
        _apply_body,
        grid=(400,),
        in_specs=[_rows(2000, _H2), _rows(2000, _FN), _const((_FN, _H2)),
                  _const((1, _H2)), _const((2, _H2)), _const((1, _H2)),
                  _const((1, _H2))],
        out_specs=_rows(2000, _FA),
        out_shape=jax.ShapeDtypeStruct((_E, _FA), _f32),
    )(ep, nf, wf, b2, st, g1, b1)


def _astat_body(s_ref, o_ref):
    x = s_ref[...]
    s = jnp.concatenate([jnp.sum(x, axis=0, keepdims=True),
                         jnp.sum(x * x, axis=0, keepdims=True)], axis=0)

    @pl.when(pl.program_id(0) == 0)
    def _():
        o_ref[...] = s

    @pl.when(pl.program_id(0) > 0)
    def _():
        o_ref[...] += s


def _astat(summed):
    return pl.pallas_call(
        _astat_body,
        grid=(125,),
        in_specs=[_rows(400, _FA)],
        out_specs=_const((2, _FA)),
        out_shape=jax.ShapeDtypeStruct((2, _FA), _f32),
    )(summed)


def _update_body(x_ref, s_ref, st_ref, g2_ref, b2_ref, o_ref):
    mean = st_ref[0:1, :] * (1.0 / _N)
    var = st_ref[1:2, :] * (1.0 / _N) - mean * mean
    t = x_ref[...] + (s_ref[...] - mean) * (lax.rsqrt(var + _EPS)
                                            * g2_ref[...]) + b2_ref[...]
    o_ref[...] = _softplus(t)


def _update(x, summed, st, g2, b2):
    return pl.pallas_call(
        _update_body,
        grid=(125,),
        in_specs=[_rows(400, _FA), _rows(400, _FA), _const((2, _FA)),
                  _const((1, _FA)), _const((1, _FA))],
        out_specs=_rows(400, _FA),
        out_shape=jax.ShapeDtypeStruct((_N, _FA), _f32),
    )(x, summed, st, g2, b2)


def _head_body(seg_ref, cnt_ref, wc_ref, bc_ref, wo_ref, bo_ref, o_ref):
    cnt = cnt_ref[:, 0:1]
    mean = seg_ref[...] / jnp.maximum(cnt, 1.0)
    h = _softplus(jnp.dot(_softplus(mean), wc_ref[...],
                          preferred_element_type=_f32) + bc_ref[...])
    o_ref[...] = jnp.sum(h * wo_ref[...], axis=1, keepdims=True) + bo_ref[...]


def _head(seg, cnt, wc, bc2, woT, bo2):
    return pl.pallas_call(
        _head_body,
        grid=(1,),
        in_specs=[_const((_C, _FA)), _const((_C, _FA)), _const((_FA, 128)),
                  _const((1, 128)), _const((1, 128)), _const((1, 1))],
        out_specs=_const((_C, 1)),
        out_shape=jax.ShapeDtypeStruct((_C, 1), _f32),
    )(seg, cnt, wc, bc2, woT, bo2)


_NC, _NS, _L = 2, 16, 16
_NW = _NC * _NS
_CH = 128
_G_NCHUNK = _E // _CH
_G_ITERS = -(-_G_NCHUNK // _NW)
_S_ITERS = -(-_G_NCHUNK // _NS)
_HALF = _N // 2
_ACC_R = 25600
_S_FULLC = _HALF // _CH
_S_TAIL = _HALF - _S_FULLC * _CH
_P_NCHUNK = _N // _CH
_P_TAIL = _N - _P_NCHUNK * _CH
_CHALF = _C // 2


def _sc_gather_body(si_hbm, ni_hbm, xs_hbm, xn_hbm, out_hbm,
                    isv, inv, A, B, s1, s2):
    wid = lax.axis_index("s") * _NC + lax.axis_index("c")

    def step(i, _):
        k = wid + i * _NW

        @pl.when(k < _G_NCHUNK)
        def _():
            base = k * _CH
            pltpu.sync_copy(si_hbm.at[pl.ds(base, _CH)], isv)
            pltpu.sync_copy(ni_hbm.at[pl.ds(base, _CH)], inv)
            ca = pltpu.async_copy(xs_hbm.at[isv], A, s1)
            cb = pltpu.async_copy(xn_hbm.at[inv], B, s2)
            ca.wait()
            cb.wait()

            def addrow(r, _):
                for q in range(8):
                    sl = pl.ds(q * _L, _L)
                    A[r, sl] = A[r, sl] + B[r, sl]
                return 0

            lax.fori_loop(0, _CH, addrow, 0)
            pltpu.sync_copy(A, out_hbm.at[pl.ds(base, _CH)])
        return 0

    lax.fori_loop(0, _G_ITERS, step, 0)


def _gather(self_idx, nbr_idx, xs, xn):
    f = pl.kernel(
        _sc_gather_body,
        out_type=jax.ShapeDtypeStruct((_E, _H2), _f32),
        mesh=plsc.VectorSubcoreMesh(core_axis_name="c", subcore_axis_name="s"),
        scratch_types=[pltpu.VMEM((_CH,), _i32), pltpu.VMEM((_CH,), _i32),
                       pltpu.VMEM((_CH, _H2), _f32),
                       pltpu.VMEM((_CH, _H2), _f32),
                       pltpu.SemaphoreType.DMA, pltpu.SemaphoreType.DMA],
    )
    return f(self_idx, nbr_idx, xs, xn)


def _sc_scatter_body(si_hbm, msg_hbm, out_hbm, idxv, Mbuf, acc):
    c = lax.axis_index("c")
    s = lax.axis_index("s")
    lo = c * _HALF

    def zrow(r, _):
        for q in range(4):
            Mbuf[r, pl.ds(q * _L, _L)] = jnp.zeros((_L,), _f32)
        return 0

    lax.fori_loop(0, _CH, zrow, 0)

    def zc(q, _):
        pltpu.sync_copy(Mbuf, acc.at[pl.ds(s * 1600 + q * _CH, _CH)])
        return 0

    lax.fori_loop(0, 12, zc, 0)
    pltpu.sync_copy(Mbuf.at[pl.ds(0, 64)],
                    acc.at[pl.ds(s * 1600 + 12 * _CH, 64)])
    plsc.subcore_barrier()

    def step(i, _):
        k = s + i * _NS

        @pl.when(k < _G_NCHUNK)
        def _():
            base = k * _CH
            pltpu.sync_copy(si_hbm.at[pl.ds(base, _CH)], idxv)
            pltpu.sync_copy(msg_hbm.at[pl.ds(base, _CH)], Mbuf)

            def adj(g, _):
                sl = pl.ds(g * _L, _L)
                rel = idxv[sl] - lo
                ok = (rel >= 0) & (rel < _HALF)
                idxv[sl] = jnp.where(ok, rel, _HALF)
                return 0

            lax.fori_loop(0, _CH // _L, adj, 0)
            pltpu.sync_copy(Mbuf, acc.at[idxv], add=True)
        return 0

    lax.fori_loop(0, _S_ITERS, step, 0)
    plsc.subcore_barrier()

    def cp(i, _):
        k = s + i * _NS

        @pl.when(k < _S_FULLC)
        def _():
            pltpu.sync_copy(acc.at[pl.ds(k * _CH, _CH)],
                            out_hbm.at[pl.ds(lo + k * _CH, _CH)])
        return 0

    lax.fori_loop(0, -(-_S_FULLC // _NS), cp, 0)

    @pl.when(s == 0)
    def _():
        pltpu.sync_copy(acc.at[pl.ds(_S_FULLC * _CH, _S_TAIL)],
                        out_hbm.at[pl.ds(lo + _S_FULLC * _CH, _S_TAIL)])


def _scatter(self_idx, msg):
    f = pl.kernel(
        _sc_scatter_body,
        out_type=jax.ShapeDtypeStruct((_N, _FA), _f32),
        mesh=plsc.VectorSubcoreMesh(core_axis_name="c", subcore_axis_name="s"),
        scratch_types=[pltpu.VMEM((_CH,), _i32),
                       pltpu.VMEM((_CH, _FA), _f32),
                       pltpu.VMEM_SHARED((_ACC_R, _FA), _f32)],
    )
    return f(self_idx, msg)


def _sc_pool_body(x_hbm, ci_hbm, seg_hbm, cnt_hbm, idxv, Xbuf, Obuf,
                  sacc, cacc):
    c = lax.axis_index("c")
    s = lax.axis_index("s")
    lo = c * _CHALF

    def orow(r, _):
        for q in range(4):
            Obuf[r, pl.ds(q * _L, _L)] = jnp.ones((_L,), _f32)
            Xbuf[r, pl.ds(q * _L, _L)] = jnp.zeros((_L,), _f32)
        return 0

    lax.fori_loop(0, _CH, orow, 0)

    @pl.when(s == 0)
    def _():
        pltpu.sync_copy(Xbuf, sacc.at[pl.ds(0, _CH)])
        pltpu.sync_copy(Xbuf.at[pl.ds(0, 8)], sacc.at[pl.ds(_CH, 8)])
        pltpu.sync_copy(Xbuf, cacc.at[pl.ds(0, _CH)])
        pltpu.sync_copy(Xbuf.at[pl.ds(0, 8)], cacc.at[pl.ds(_CH, 8)])

    plsc.subcore_barrier()

    def adj(g, _):
        sl = pl.ds(g * _L, _L)
        rel = idxv[sl] - lo
        ok = (rel >= 0) & (rel < _CHALF)
        idxv[sl] = jnp.where(ok, rel, _CHALF)
        return 0

    def step(i, _):
        k = s + i * _NS

        @pl.when(k < _P_NCHUNK)
        def _():
            base = k * _CH
            pltpu.sync_copy(ci_hbm.at[pl.ds(base, _CH)], idxv)
            pltpu.sync_copy(x_hbm.at[pl.ds(base, _CH)], Xbuf)
            lax.fori_loop(0, _CH // _L, adj, 0)
            pltpu.sync_copy(Xbuf, sacc.at[idxv], add=True)
            pltpu.sync_copy(Obuf, cacc.at[idxv], add=True)
        return 0

    lax.fori_loop(0, -(-_P_NCHUNK // _NS), step, 0)

    @pl.when(s == 0)
    def _():
        base = _P_NCHUNK * _CH
        pltpu.sync_copy(ci_hbm.at[pl.ds(base, _P_TAIL)],
                        idxv.at[pl.ds(0, _P_TAIL)])
        for g in range(_P_TAIL // _L, _CH // _L):
            idxv[pl.ds(g * _L, _L)] = jnp.full((_L,), _CHALF, _i32)
        lax.fori_loop(0, _P_TAIL // _L, adj, 0)
        pltpu.sync_copy(x_hbm.at[pl.ds(base, _P_TAIL)],
                        Xbuf.at[pl.ds(0, _P_TAIL)])
        pltpu.sync_copy(Xbuf, sacc.at[idxv], add=True)
        pltpu.sync_copy(Obuf, cacc.at[idxv], add=True)

    plsc.subcore_barrier()
    pltpu.sync_copy(sacc.at[pl.ds(s * 8, 8)],
                    seg_hbm.at[pl.ds(lo + s * 8, 8)])
    pltpu.sync_copy(cacc.at[pl.ds(s * 8, 8)],
                    cnt_hbm.at[pl.ds(lo + s * 8, 8)])


def _pool(x, crystal_idx):
    f = pl.kernel(
        _sc_pool_body,
        out_type=[jax.ShapeDtypeStruct((_C, _FA), _f32),
                  jax.ShapeDtypeStruct((_C, _FA), _f32)],
        mesh=plsc.VectorSubcoreMesh(core_axis_name="c", subcore_axis_name="s"),
        scratch_types=[pltpu.VMEM((_CH,), _i32),
                       pltpu.VMEM((_CH, _FA), _f32),
                       pltpu.VMEM((_CH, _FA), _f32),
                       pltpu.VMEM_SHARED((_CHALF + 8, _FA), _f32),
                       pltpu.VMEM_SHARED((_CHALF + 8, _FA), _f32)],
    )
    return f(x, crystal_idx)


def kernel(atom_fea, nbr_fea, self_fea_idx, nbr_fea_idx, crystal_atom_idx,
           W_emb, b_emb, conv_fc_W, conv_fc_b, bn1_g, bn1_b, bn2_g, bn2_b,
           W_c2f, b_c2f, W_out, b_out):
    self_fea_idx = self_fea_idx.astype(_i32)
    nbr_fea_idx = nbr_fea_idx.astype(_i32)
    crystal_atom_idx = crystal_atom_idx.astype(_i32)

    x = _embed(atom_fea, W_emb, b_emb.reshape(1, _FA))
    for i in range(3):
        W = conv_fc_W[i]
        b2 = conv_fc_b[i].reshape(1, _H2)
        xs, xn = _atom_mm(x, W[0:_FA], W[_FA:2 * _FA])
        ep = _gather(self_fea_idx, nbr_fea_idx, xs, xn)
        wf = W[2 * _FA:]
        st = _stats(ep, nbr_fea, wf, b2)
        msg = _apply(ep, nbr_fea, wf, b2, st,
                     bn1_g[i].reshape(1, _H2), bn1_b[i].reshape(1, _H2))
        summed = _scatter(self_fea_idx, msg)
        ast = _astat(summed)
        x = _update(x, summed, ast,
                    bn2_g[i].reshape(1, _FA), bn2_b[i].reshape(1, _FA))
    seg, cnt = _pool(x, crystal_atom_idx)
    return _head(seg, cnt, W_c2f, b_c2f.reshape(1, 128),
                 W_out.reshape(1, 128), b_out.reshape(1, 1))


# speedup vs baseline: 1.5805x; 1.5805x over previous
import jax
import jax.numpy as jnp
from jax import lax
from jax.experimental import pallas as pl
from jax.experimental.pallas import tpu as pltpu
from jax.experimental.pallas import tpu_sc as plsc

_N = 50000
_E = 800000
_C = 256
_FA = 64
_FN = 16
_H2 = 128
_EPS = 1e-5
_f32 = jnp.float32
_i32 = jnp.int32


def _softplus(x):
    return jnp.maximum(x, 0.0) + jnp.log(1.0 + jnp.exp(-jnp.abs(x)))


def _sigmoid(x):
    return 1.0 / (1.0 + jnp.exp(-x))


def _rows(block_rows, width):
    return pl.BlockSpec((block_rows, width), lambda i: (i, 0))


def _const(shape):
    return pl.BlockSpec(shape, lambda i: tuple(0 for _ in shape))


def _embed_body(a_ref, w_ref, b_ref, o_ref):
    o_ref[...] = jnp.dot(a_ref[...], w_ref[...],
                         preferred_element_type=_f32) + b_ref[...]


def _embed(atom_fea, w, b2):
    return pl.pallas_call(
        _embed_body,
        grid=(125,),
        in_specs=[_rows(400, 128), _const((128, 64)), _const((1, 64))],
        out_specs=_rows(400, 64),
        out_shape=jax.ShapeDtypeStruct((_N, _FA), _f32),
    )(atom_fea, w, b2)


def _atom_mm_body(x_ref, ws_ref, wn_ref, xs_ref, xn_ref):
    x = x_ref[...]
    xs_ref[...] = jnp.dot(x, ws_ref[...], preferred_element_type=_f32)
    xn_ref[...] = jnp.dot(x, wn_ref[...], preferred_element_type=_f32)


def _atom_mm(x, ws, wn):
    return pl.pallas_call(
        _atom_mm_body,
        grid=(125,),
        in_specs=[_rows(400, _FA), _const((_FA, _H2)), _const((_FA, _H2))],
        out_specs=[_rows(400, _H2), _rows(400, _H2)],
        out_shape=[jax.ShapeDtypeStruct((_N, _H2), _f32),
                   jax.ShapeDtypeStruct((_N, _H2), _f32)],
    )(x, ws, wn)


def _stats_body(ep_ref, nf_ref, wf_ref, b_ref, o_ref):
    e = ep_ref[...] + jnp.dot(nf_ref[...], wf_ref[...],
                              preferred_element_type=_f32) + b_ref[...]
    s = jnp.concatenate([jnp.sum(e, axis=0, keepdims=True),
                         jnp.sum(e * e, axis=0, keepdims=True)], axis=0)

    @pl.when(pl.program_id(0) == 0)
    def _():
        o_ref[...] = s

    @pl.when(pl.program_id(0) > 0)
    def _():
        o_ref[...] += s


def _stats(ep, nf, wf, b2):
    return pl.pallas_call(
        _stats_body,
        grid=(400,),
        in_specs=[_rows(2000, _H2), _rows(2000, _FN),
                  _const((_FN, _H2)), _const((1, _H2))],
        out_specs=_const((2, _H2)),
        out_shape=jax.ShapeDtypeStruct((2, _H2), _f32),
    )(ep, nf, wf, b2)


def _apply_body(ep_ref, nf_ref, wf_ref, b_ref, st_ref, g1_ref, b1_ref,
                par_ref, o_ref):
    e = ep_ref[...] + jnp.dot(nf_ref[...], wf_ref[...],
                              preferred_element_type=_f32) + b_ref[...]
    mean = st_ref[0:1, :] * (1.0 / _E)
    var = st_ref[1:2, :] * (1.0 / _E) - mean * mean
    ebn = (e - mean) * (lax.rsqrt(var + _EPS) * g1_ref[...]) + b1_ref[...]
    m = _sigmoid(ebn[:, :_FA]) * _softplus(ebn[:, _FA:])
    p = par_ref[...]
    o_ref[...] = jnp.concatenate([m * (1.0 - p), m * p], axis=1)


def _apply(ep, nf, wf, b2, st, g1, b1, par):
    return pl.pallas_call(
        _apply_body,
        grid=(400,),
        in_specs=[_rows(2000, _H2), _rows(2000, _FN), _const((_FN, _H2)),
                  _const((1, _H2)), _const((2, _H2)), _const((1, _H2)),
                  _const((1, _H2)), _rows(2000, 1)],
        out_specs=_rows(2000, _H2),
        out_shape=jax.ShapeDtypeStruct((_E, _H2), _f32),
    )(ep, nf, wf, b2, st, g1, b1, par)


def _astat_body(s_ref, o_ref):
    x = s_ref[...]
    s = jnp.concatenate([jnp.sum(x, axis=0, keepdims=True),
                         jnp.sum(x * x, axis=0, keepdims=True)], axis=0)

    @pl.when(pl.program_id(0) == 0)
    def _():
        o_ref[...] = s

    @pl.when(pl.program_id(0) > 0)
    def _():
        o_ref[...] += s


def _astat(summed):
    return pl.pallas_call(
        _astat_body,
        grid=(125,),
        in_specs=[_rows(400, _FA)],
        out_specs=_const((2, _FA)),
        out_shape=jax.ShapeDtypeStruct((2, _FA), _f32),
    )(summed)


def _update_body(x_ref, s_ref, st_ref, g2_ref, b2_ref, o_ref):
    mean = st_ref[0:1, :] * (1.0 / _N)
    var = st_ref[1:2, :] * (1.0 / _N) - mean * mean
    t = x_ref[...] + (s_ref[...] - mean) * (lax.rsqrt(var + _EPS)
                                            * g2_ref[...]) + b2_ref[...]
    o_ref[...] = _softplus(t)


def _update(x, summed, st, g2, b2):
    return pl.pallas_call(
        _update_body,
        grid=(125,),
        in_specs=[_rows(400, _FA), _rows(400, _FA), _const((2, _FA)),
                  _const((1, _FA)), _const((1, _FA))],
        out_specs=_rows(400, _FA),
        out_shape=jax.ShapeDtypeStruct((_N, _FA), _f32),
    )(x, summed, st, g2, b2)


def _mk_prep_body(half, garbage, shift):
    def body(i_ref, o_ref):
        v = i_ref[...]
        r1 = v - half
        o_ref[0] = jnp.where((v >= 0) & (v < half), v >> shift, garbage)
        o_ref[1] = jnp.where((r1 >= 0) & (r1 < half), r1 >> shift, garbage)
    return body


def _prep(idx2d, half, garbage, grid, blk, shift):
    rows = idx2d.shape[0]
    return pl.pallas_call(
        _mk_prep_body(half, garbage, shift),
        grid=(grid,),
        in_specs=[pl.BlockSpec((blk, 256), lambda i: (i, 0))],
        out_specs=pl.BlockSpec((2, blk, 256), lambda i: (0, i, 0)),
        out_shape=jax.ShapeDtypeStruct((2, rows, 256), _i32),
    )(idx2d)


def _padx_body(x_ref, o_ref):
    x = x_ref[...]
    o_ref[...] = jnp.concatenate([x, jnp.zeros_like(x)], axis=1)


def _padx(x):
    return pl.pallas_call(
        _padx_body,
        grid=(125,),
        in_specs=[_rows(400, _FA)],
        out_specs=_rows(400, _H2),
        out_shape=jax.ShapeDtypeStruct((_N, _H2), _f32),
    )(x)


def _head_body(seg_ref, cnt_ref, wc_ref, bc_ref, wo_ref, bo_ref, o_ref):
    cnt = cnt_ref[:, 0:1]
    mean = seg_ref[:, :_FA] / jnp.maximum(cnt, 1.0)
    h = _softplus(jnp.dot(_softplus(mean), wc_ref[...],
                          preferred_element_type=_f32) + bc_ref[...])
    o_ref[...] = jnp.sum(h * wo_ref[...], axis=1, keepdims=True) + bo_ref[...]


def _head(seg, cnt, wc, bc2, woT, bo2):
    return pl.pallas_call(
        _head_body,
        grid=(1,),
        in_specs=[_const((_C, _H2)), _const((_C, _H2)), _const((_FA, 128)),
                  _const((1, 128)), _const((1, 128)), _const((1, 1))],
        out_specs=_const((_C, 1)),
        out_shape=jax.ShapeDtypeStruct((_C, 1), _f32),
    )(seg, cnt, wc, bc2, woT, bo2)


_NC, _NS, _L = 2, 16, 16
_NW = _NC * _NS
_CH = 128
_G_NCHUNK = _E // _CH
_G_ITERS = -(-_G_NCHUNK // _NW)
_S_ITERS = -(-_G_NCHUNK // _NS)
_HALF = _N // 2
_PAIRS = _HALF // 2
_GARB = 12600
_ACC_R = 12800
_S_FULLC = _PAIRS // _CH
_S_TAIL = _PAIRS - _S_FULLC * _CH
_P_NCHUNK = _N // _CH
_P_TAIL = _N - _P_NCHUNK * _CH
_CHALF = _C // 2


def _sc_gather_body(si_hbm, ni_hbm, xs_hbm, xn_hbm, out_hbm,
                    isv, inv, A, B, s1, s2):
    wid = lax.axis_index("s") * _NC + lax.axis_index("c")

    def step(i, _):
        k = wid + i * _NW

        @pl.when(k < _G_NCHUNK)
        def _():
            base = k * _CH
            pltpu.sync_copy(si_hbm.at[pl.ds(base, _CH)], isv)
            pltpu.sync_copy(ni_hbm.at[pl.ds(base, _CH)], inv)
            ca = pltpu.async_copy(xs_hbm.at[isv], A, s1)
            cb = pltpu.async_copy(xn_hbm.at[inv], B, s2)
            ca.wait()
            cb.wait()

            def addrow(r, _):
                for q in range(8):
                    sl = pl.ds(q * _L, _L)
                    A[r, sl] = A[r, sl] + B[r, sl]
                return 0

            lax.fori_loop(0, _CH, addrow, 0)
            pltpu.sync_copy(A, out_hbm.at[pl.ds(base, _CH)])
        return 0

    lax.fori_loop(0, _G_ITERS, step, 0)


def _gather(self_idx, nbr_idx, xs, xn):
    f = pl.kernel(
        _sc_gather_body,
        out_type=jax.ShapeDtypeStruct((_E, _H2), _f32),
        mesh=plsc.VectorSubcoreMesh(core_axis_name="c", subcore_axis_name="s"),
        scratch_types=[pltpu.VMEM((_CH,), _i32), pltpu.VMEM((_CH,), _i32),
                       pltpu.VMEM((_CH, _H2), _f32),
                       pltpu.VMEM((_CH, _H2), _f32),
                       pltpu.SemaphoreType.DMA, pltpu.SemaphoreType.DMA],
    )
    return f(self_idx, nbr_idx, xs, xn)


def _sc_scatter_body(si2_hbm, msg_hbm, out_hbm, idxv, Mbuf, acc):
    c = lax.axis_index("c")
    s = lax.axis_index("s")

    def zrow(r, _):
        for q in range(8):
            Mbuf[r, pl.ds(q * _L, _L)] = jnp.zeros((_L,), _f32)
        return 0

    lax.fori_loop(0, _CH, zrow, 0)

    def zc(q, _):
        pltpu.sync_copy(Mbuf, acc.at[pl.ds(s * 800 + q * _CH, _CH)])
        return 0

    lax.fori_loop(0, 6, zc, 0)
    pltpu.sync_copy(Mbuf.at[pl.ds(0, 32)],
                    acc.at[pl.ds(s * 800 + 6 * _CH, 32)])
    plsc.subcore_barrier()

    def step(i, _):
        k = s + i * _NS

        @pl.when(k < _G_NCHUNK)
        def _():
            base = k * _CH
            pltpu.sync_copy(si2_hbm.at[c, pl.ds(base, _CH)], idxv)
            pltpu.sync_copy(msg_hbm.at[pl.ds(base, _CH)], Mbuf)
            pltpu.sync_copy(Mbuf, acc.at[idxv], add=True)
        return 0

    lax.fori_loop(0, _S_ITERS, step, 0)
    plsc.subcore_barrier()

    def cp(i, _):
        k = s + i * _NS

        @pl.when(k < _S_FULLC)
        def _():
            pltpu.sync_copy(acc.at[pl.ds(k * _CH, _CH)],
                            out_hbm.at[c, pl.ds(k * _CH, _CH)])
        return 0

    lax.fori_loop(0, -(-_S_FULLC // _NS), cp, 0)

    @pl.when(s == 0)
    def _():
        pltpu.sync_copy(acc.at[pl.ds(_S_FULLC * _CH, _S_TAIL)],
                        out_hbm.at[c, pl.ds(_S_FULLC * _CH, _S_TAIL)])


def _scatter(si2, msg2):
    f = pl.kernel(
        _sc_scatter_body,
        out_type=jax.ShapeDtypeStruct((2, _PAIRS, _H2), _f32),
        mesh=plsc.VectorSubcoreMesh(core_axis_name="c", subcore_axis_name="s"),
        scratch_types=[pltpu.VMEM((_CH,), _i32),
                       pltpu.VMEM((_CH, _H2), _f32),
                       pltpu.VMEM_SHARED((_ACC_R, _H2), _f32)],
    )
    return f(si2, msg2).reshape(_N, _FA)


def _sc_pool_body(x_hbm, ci2_hbm, seg_hbm, cnt_hbm, idxv, Xbuf, Obuf,
                  sacc, cacc):
    c = lax.axis_index("c")
    s = lax.axis_index("s")
    lo = c * _CHALF

    def orow(r, _):
        for q in range(4):
            Obuf[r, pl.ds(q * _L, _L)] = jnp.ones((_L,), _f32)
            Obuf[r, pl.ds((q + 4) * _L, _L)] = jnp.zeros((_L,), _f32)
        for q in range(8):
            Xbuf[r, pl.ds(q * _L, _L)] = jnp.zeros((_L,), _f32)
        return 0

    lax.fori_loop(0, _CH, orow, 0)

    @pl.when(s == 0)
    def _():
        pltpu.sync_copy(Xbuf, sacc.at[pl.ds(0, _CH)])
        pltpu.sync_copy(Xbuf.at[pl.ds(0, 8)], sacc.at[pl.ds(_CH, 8)])
        pltpu.sync_copy(Xbuf, cacc.at[pl.ds(0, _CH)])
        pltpu.sync_copy(Xbuf.at[pl.ds(0, 8)], cacc.at[pl.ds(_CH, 8)])

    plsc.subcore_barrier()

    def step(i, _):
        k = s + i * _NS

        @pl.when(k < _P_NCHUNK)
        def _():
            base = k * _CH
            pltpu.sync_copy(ci2_hbm.at[c, pl.ds(base, _CH)], idxv)
            pltpu.sync_copy(x_hbm.at[pl.ds(base, _CH)], Xbuf)
            pltpu.sync_copy(Xbuf, sacc.at[idxv], add=True)
            pltpu.sync_copy(Obuf, cacc.at[idxv], add=True)
        return 0

    lax.fori_loop(0, -(-_P_NCHUNK // _NS), step, 0)

    @pl.when(s == 0)
    def _():
        base = _P_NCHUNK * _CH
        pltpu.sync_copy(ci2_hbm.at[c, pl.ds(base, _CH)], idxv)
        pltpu.sync_copy(x_hbm.at[pl.ds(base, _P_TAIL)],
                        Xbuf.at[pl.ds(0, _P_TAIL)])
        pltpu.sync_copy(Xbuf, sacc.at[idxv], add=True)
        pltpu.sync_copy(Obuf, cacc.at[idxv], add=True)

    plsc.subcore_barrier()
    pltpu.sync_copy(sacc.at[pl.ds(s * 8, 8)],
                    seg_hbm.at[pl.ds(lo + s * 8, 8)])
    pltpu.sync_copy(cacc.at[pl.ds(s * 8, 8)],
                    cnt_hbm.at[pl.ds(lo + s * 8, 8)])


def _pool(xp, ci2):
    f = pl.kernel(
        _sc_pool_body,
        out_type=[jax.ShapeDtypeStruct((_C, _H2), _f32),
                  jax.ShapeDtypeStruct((_C, _H2), _f32)],
        mesh=plsc.VectorSubcoreMesh(core_axis_name="c", subcore_axis_name="s"),
        scratch_types=[pltpu.VMEM((_CH,), _i32),
                       pltpu.VMEM((_CH, _H2), _f32),
                       pltpu.VMEM((_CH, _H2), _f32),
                       pltpu.VMEM_SHARED((_CHALF + 8, _H2), _f32),
                       pltpu.VMEM_SHARED((_CHALF + 8, _H2), _f32)],
    )
    return f(xp, ci2)


def kernel(atom_fea, nbr_fea, self_fea_idx, nbr_fea_idx, crystal_atom_idx,
           W_emb, b_emb, conv_fc_W, conv_fc_b, bn1_g, bn1_b, bn2_g, bn2_b,
           W_c2f, b_c2f, W_out, b_out):
    self_fea_idx = self_fea_idx.astype(_i32)
    nbr_fea_idx = nbr_fea_idx.astype(_i32)
    crystal_atom_idx = crystal_atom_idx.astype(_i32)

    si_pad = jnp.pad(self_fea_idx, (0, 3200 * 256 - _E),
                     constant_values=_N).reshape(3200, 256)
    si2 = _prep(si_pad, _HALF, _GARB, 25, 128, 1).reshape(2, 3200 * 256)
    ci_pad = jnp.pad(crystal_atom_idx, (0, 200 * 256 - _N),
                     constant_values=_N).reshape(200, 256)
    ci2 = _prep(ci_pad, _CHALF, _CHALF, 25, 8, 0).reshape(2, 200 * 256)
    par = (self_fea_idx & 1).astype(_f32).reshape(_E, 1)

    x = _embed(atom_fea, W_emb, b_emb.reshape(1, _FA))
    for i in range(3):
        W = conv_fc_W[i]
        b2 = conv_fc_b[i].reshape(1, _H2)
        xs, xn = _atom_mm(x, W[0:_FA], W[_FA:2 * _FA])
        ep = _gather(self_fea_idx, nbr_fea_idx, xs, xn)
        wf = W[2 * _FA:]
        st = _stats(ep, nbr_fea, wf, b2)
        msg2 = _apply(ep, nbr_fea, wf, b2, st,
                      bn1_g[i].reshape(1, _H2), bn1_b[i].reshape(1, _H2), par)
        summed = _scatter(si2, msg2)
        ast = _astat(summed)
        x = _update(x, summed, ast,
                    bn2_g[i].reshape(1, _FA), bn2_b[i].reshape(1, _FA))
    seg, cnt = _pool(_padx(x), ci2)
    return _head(seg, cnt, W_c2f, b_c2f.reshape(1, 128),
                 W_out.reshape(1, 128), b_out.reshape(1, 1))


# speedup vs baseline: 1.8171x; 1.1497x over previous
import jax
import jax.numpy as jnp
from jax import lax
from jax.experimental import pallas as pl
from jax.experimental.pallas import tpu as pltpu
from jax.experimental.pallas import tpu_sc as plsc

_N = 50000
_E = 800000
_C = 256
_FA = 64
_FN = 16
_H2 = 128
_EPS = 1e-5
_f32 = jnp.float32
_i32 = jnp.int32


def _softplus(x):
    return jnp.maximum(x, 0.0) + jnp.log(1.0 + jnp.exp(-jnp.abs(x)))


def _sigmoid(x):
    return 1.0 / (1.0 + jnp.exp(-x))


def _rows(block_rows, width):
    return pl.BlockSpec((block_rows, width), lambda i: (i, 0))


def _const(shape):
    return pl.BlockSpec(shape, lambda i: tuple(0 for _ in shape))


def _embed_body(a_ref, w_ref, b_ref, o_ref):
    o_ref[...] = jnp.dot(a_ref[...], w_ref[...],
                         preferred_element_type=_f32) + b_ref[...]


def _embed(atom_fea, w, b2):
    return pl.pallas_call(
        _embed_body,
        grid=(125,),
        in_specs=[_rows(400, 128), _const((128, 64)), _const((1, 64))],
        out_specs=_rows(400, 64),
        out_shape=jax.ShapeDtypeStruct((_N, _FA), _f32),
    )(atom_fea, w, b2)


def _atom_mm_body(x_ref, ws_ref, wn_ref, xs_ref, xn_ref):
    x = x_ref[...]
    xs_ref[...] = jnp.dot(x, ws_ref[...], preferred_element_type=_f32)
    xn_ref[...] = jnp.dot(x, wn_ref[...], preferred_element_type=_f32)


def _atom_mm(x, ws, wn):
    return pl.pallas_call(
        _atom_mm_body,
        grid=(125,),
        in_specs=[_rows(400, _FA), _const((_FA, _H2)), _const((_FA, _H2))],
        out_specs=[_rows(400, _H2), _rows(400, _H2)],
        out_shape=[jax.ShapeDtypeStruct((_N, _H2), _f32),
                   jax.ShapeDtypeStruct((_N, _H2), _f32)],
    )(x, ws, wn)


def _stats_body(ep_ref, nf_ref, wf_ref, b_ref, o_ref):
    e = ep_ref[...] + jnp.dot(nf_ref[...], wf_ref[...],
                              preferred_element_type=_f32) + b_ref[...]
    s = jnp.concatenate([jnp.sum(e, axis=0, keepdims=True),
                         jnp.sum(e * e, axis=0, keepdims=True)], axis=0)

    @pl.when(pl.program_id(0) == 0)
    def _():
        o_ref[...] = s

    @pl.when(pl.program_id(0) > 0)
    def _():
        o_ref[...] += s


def _stats(ep, nf, wf, b2):
    return pl.pallas_call(
        _stats_body,
        grid=(400,),
        in_specs=[_rows(2000, _H2), _rows(2000, _FN),
                  _const((_FN, _H2)), _const((1, _H2))],
        out_specs=_const((2, _H2)),
        out_shape=jax.ShapeDtypeStruct((2, _H2), _f32),
    )(ep, nf, wf, b2)


def _apply_body(ep_ref, nf_ref, wf_ref, b_ref, st_ref, g1_ref, b1_ref,
                par_ref, o_ref):
    e = ep_ref[...] + jnp.dot(nf_ref[...], wf_ref[...],
                              preferred_element_type=_f32) + b_ref[...]
    mean = st_ref[0:1, :] * (1.0 / _E)
    var = st_ref[1:2, :] * (1.0 / _E) - mean * mean
    ebn = (e - mean) * (lax.rsqrt(var + _EPS) * g1_ref[...]) + b1_ref[...]
    m = _sigmoid(ebn[:, :_FA]) * _softplus(ebn[:, _FA:])
    p = par_ref[...]
    o_ref[...] = jnp.concatenate([m * (1.0 - p), m * p], axis=1)


def _apply(ep, nf, wf, b2, st, g1, b1, par):
    return pl.pallas_call(
        _apply_body,
        grid=(400,),
        in_specs=[_rows(2000, _H2), _rows(2000, _FN), _const((_FN, _H2)),
                  _const((1, _H2)), _const((2, _H2)), _const((1, _H2)),
                  _const((1, _H2)), _rows(2000, 1)],
        out_specs=_rows(2000, _H2),
        out_shape=jax.ShapeDtypeStruct((_E, _H2), _f32),
    )(ep, nf, wf, b2, st, g1, b1, par)


def _astat_body(s_ref, o_ref):
    x = s_ref[...]
    s = jnp.concatenate([jnp.sum(x, axis=0, keepdims=True),
                         jnp.sum(x * x, axis=0, keepdims=True)], axis=0)

    @pl.when(pl.program_id(0) == 0)
    def _():
        o_ref[...] = s

    @pl.when(pl.program_id(0) > 0)
    def _():
        o_ref[...] += s


def _astat(summed):
    return pl.pallas_call(
        _astat_body,
        grid=(125,),
        in_specs=[_rows(400, _FA)],
        out_specs=_const((2, _FA)),
        out_shape=jax.ShapeDtypeStruct((2, _FA), _f32),
    )(summed)


def _update_body(x_ref, s_ref, st_ref, g2_ref, b2_ref, o_ref):
    mean = st_ref[0:1, :] * (1.0 / _N)
    var = st_ref[1:2, :] * (1.0 / _N) - mean * mean
    t = x_ref[...] + (s_ref[...] - mean) * (lax.rsqrt(var + _EPS)
                                            * g2_ref[...]) + b2_ref[...]
    o_ref[...] = _softplus(t)


def _update(x, summed, st, g2, b2):
    return pl.pallas_call(
        _update_body,
        grid=(125,),
        in_specs=[_rows(400, _FA), _rows(400, _FA), _const((2, _FA)),
                  _const((1, _FA)), _const((1, _FA))],
        out_specs=_rows(400, _FA),
        out_shape=jax.ShapeDtypeStruct((_N, _FA), _f32),
    )(x, summed, st, g2, b2)


def _mk_prep_body(half, garbage, shift):
    def body(i_ref, o_ref):
        v = i_ref[...]
        r1 = v - half
        o_ref[0] = jnp.where((v >= 0) & (v < half), v >> shift, garbage)
        o_ref[1] = jnp.where((r1 >= 0) & (r1 < half), r1 >> shift, garbage)
    return body


def _prep(idx2d, half, garbage, grid, blk, shift):
    rows = idx2d.shape[0]
    return pl.pallas_call(
        _mk_prep_body(half, garbage, shift),
        grid=(grid,),
        in_specs=[pl.BlockSpec((blk, 256), lambda i: (i, 0))],
        out_specs=pl.BlockSpec((2, blk, 256), lambda i: (0, i, 0)),
        out_shape=jax.ShapeDtypeStruct((2, rows, 256), _i32),
    )(idx2d)


def _padx_body(x_ref, o_ref):
    x = x_ref[...]
    o_ref[...] = jnp.concatenate([x, jnp.zeros_like(x)], axis=1)


def _padx(x):
    return pl.pallas_call(
        _padx_body,
        grid=(125,),
        in_specs=[_rows(400, _FA)],
        out_specs=_rows(400, _H2),
        out_shape=jax.ShapeDtypeStruct((_N, _H2), _f32),
    )(x)


def _head_body(seg_ref, cnt_ref, wc_ref, bc_ref, wo_ref, bo_ref, o_ref):
    cnt = cnt_ref[:, 0:1]
    mean = seg_ref[:, :_FA] / jnp.maximum(cnt, 1.0)
    h = _softplus(jnp.dot(_softplus(mean), wc_ref[...],
                          preferred_element_type=_f32) + bc_ref[...])
    o_ref[...] = jnp.sum(h * wo_ref[...], axis=1, keepdims=True) + bo_ref[...]


def _head(seg, cnt, wc, bc2, woT, bo2):
    return pl.pallas_call(
        _head_body,
        grid=(1,),
        in_specs=[_const((_C, _H2)), _const((_C, _H2)), _const((_FA, 128)),
                  _const((1, 128)), _const((1, 128)), _const((1, 1))],
        out_specs=_const((_C, 1)),
        out_shape=jax.ShapeDtypeStruct((_C, 1), _f32),
    )(seg, cnt, wc, bc2, woT, bo2)


_NC, _NS, _L = 2, 16, 16
_NW = _NC * _NS
_CH = 128
_G_NCHUNK = _E // _CH
_G_ITERS = -(-_G_NCHUNK // _NW)
_S_ITERS = -(-_G_NCHUNK // _NS)
_HALF = _N // 2
_PAIRS = _HALF // 2
_GARB = 12600
_ACC_R = 12800
_S_FULLC = _PAIRS // _CH
_S_TAIL = _PAIRS - _S_FULLC * _CH
_P_NCHUNK = _N // _CH
_P_TAIL = _N - _P_NCHUNK * _CH
_CHALF = _C // 2


_GCH = 64
_G_NCH = _E // _GCH
_NSLOT = 4


def _sc_gather_body(si_hbm, ni_hbm, xs_hbm, xn_hbm, out_hbm,
                    IS, IN, A, B, semi, semg, semw):
    w = lax.axis_index("s") * _NC + lax.axis_index("c")
    lo = (w * _G_NCH) >> 5
    hi = ((w + 1) * _G_NCH) >> 5

    def g_issue(r, ci):
        pltpu.async_copy(xs_hbm.at[IS.at[r]], A.at[r], semg.at[r])
        pltpu.async_copy(xn_hbm.at[IN.at[r]], B.at[r], semg.at[r])

    def g_wait(r):
        pltpu.make_async_copy(xs_hbm.at[IS.at[r]], A.at[r], semg.at[r]).wait()
        pltpu.make_async_copy(xn_hbm.at[IN.at[r]], B.at[r], semg.at[r]).wait()

    for r in range(_NSLOT):
        ci = lo + r

        @pl.when(ci < hi)
        def _(r=r, ci=ci):
            pltpu.sync_copy(si_hbm.at[pl.ds(ci * _GCH, _GCH)], IS.at[r])
            pltpu.sync_copy(ni_hbm.at[pl.ds(ci * _GCH, _GCH)], IN.at[r])
            g_issue(r, ci)

    def it(i4, _):
        for r in range(_NSLOT):
            ci = lo + i4 * _NSLOT + r
            cn = ci + _NSLOT

            @pl.when(ci < hi)
            def _(r=r, ci=ci, cn=cn):
                g_wait(r)

                @pl.when(cn < hi)
                def _():
                    pltpu.async_copy(si_hbm.at[pl.ds(cn * _GCH, _GCH)],
                                     IS.at[r], semi.at[r])
                    pltpu.async_copy(ni_hbm.at[pl.ds(cn * _GCH, _GCH)],
                                     IN.at[r], semi.at[r])

                Ar = A.at[r]
                Br = B.at[r]

                def addrow(rr, _):
                    for q in range(8):
                        sl = pl.ds(q * _L, _L)
                        Ar[rr, sl] = Ar[rr, sl] + Br[rr, sl]
                    return 0

                lax.fori_loop(0, _GCH, addrow, 0)
                pltpu.async_copy(A.at[r], out_hbm.at[pl.ds(ci * _GCH, _GCH)],
                                 semw.at[r])

                @pl.when(cn < hi)
                def _():
                    pltpu.make_async_copy(si_hbm.at[pl.ds(cn * _GCH, _GCH)],
                                          IS.at[r], semi.at[r]).wait()
                    pltpu.make_async_copy(ni_hbm.at[pl.ds(cn * _GCH, _GCH)],
                                          IN.at[r], semi.at[r]).wait()
                    pltpu.make_async_copy(
                        A.at[r], out_hbm.at[pl.ds(ci * _GCH, _GCH)],
                        semw.at[r]).wait()
                    g_issue(r, cn)
        return 0

    lax.fori_loop(0, (hi - lo + _NSLOT - 1) >> 2, it, 0)

    for r in range(_NSLOT):
        @pl.when(lo + r < hi)
        def _(r=r):
            pltpu.make_async_copy(A.at[r], out_hbm.at[pl.ds(0, _GCH)],
                                  semw.at[r]).wait()


def _gather(self_idx, nbr_idx, xs, xn):
    f = pl.kernel(
        _sc_gather_body,
        out_type=jax.ShapeDtypeStruct((_E, _H2), _f32),
        mesh=plsc.VectorSubcoreMesh(core_axis_name="c", subcore_axis_name="s"),
        scratch_types=[pltpu.VMEM((_NSLOT, _GCH), _i32),
                       pltpu.VMEM((_NSLOT, _GCH), _i32),
                       pltpu.VMEM((_NSLOT, _GCH, _H2), _f32),
                       pltpu.VMEM((_NSLOT, _GCH, _H2), _f32),
                       pltpu.SemaphoreType.DMA((_NSLOT,)),
                       pltpu.SemaphoreType.DMA((_NSLOT,)),
                       pltpu.SemaphoreType.DMA((_NSLOT,))],
    )
    return f(self_idx, nbr_idx, xs, xn)


def _sc_scatter_body(si2_hbm, msg_hbm, out_hbm, idxv, Mbuf, acc):
    c = lax.axis_index("c")
    s = lax.axis_index("s")

    def zrow(r, _):
        for q in range(8):
            Mbuf[r, pl.ds(q * _L, _L)] = jnp.zeros((_L,), _f32)
        return 0

    lax.fori_loop(0, _CH, zrow, 0)

    def zc(q, _):
        pltpu.sync_copy(Mbuf, acc.at[pl.ds(s * 800 + q * _CH, _CH)])
        return 0

    lax.fori_loop(0, 6, zc, 0)
    pltpu.sync_copy(Mbuf.at[pl.ds(0, 32)],
                    acc.at[pl.ds(s * 800 + 6 * _CH, 32)])
    plsc.subcore_barrier()

    def step(i, _):
        k = s + i * _NS

        @pl.when(k < _G_NCHUNK)
        def _():
            base = k * _CH
            pltpu.sync_copy(si2_hbm.at[c, pl.ds(base, _CH)], idxv)
            pltpu.sync_copy(msg_hbm.at[pl.ds(base, _CH)], Mbuf)
            pltpu.sync_copy(Mbuf, acc.at[idxv], add=True)
        return 0

    lax.fori_loop(0, _S_ITERS, step, 0)
    plsc.subcore_barrier()

    def cp(i, _):
        k = s + i * _NS

        @pl.when(k < _S_FULLC)
        def _():
            pltpu.sync_copy(acc.at[pl.ds(k * _CH, _CH)],
                            out_hbm.at[c, pl.ds(k * _CH, _CH)])
        return 0

    lax.fori_loop(0, -(-_S_FULLC // _NS), cp, 0)

    @pl.when(s == 0)
    def _():
        pltpu.sync_copy(acc.at[pl.ds(_S_FULLC * _CH, _S_TAIL)],
                        out_hbm.at[c, pl.ds(_S_FULLC * _CH, _S_TAIL)])


def _scatter(si2, msg2):
    f = pl.kernel(
        _sc_scatter_body,
        out_type=jax.ShapeDtypeStruct((2, _PAIRS, _H2), _f32),
        mesh=plsc.VectorSubcoreMesh(core_axis_name="c", subcore_axis_name="s"),
        scratch_types=[pltpu.VMEM((_CH,), _i32),
                       pltpu.VMEM((_CH, _H2), _f32),
                       pltpu.VMEM_SHARED((_ACC_R, _H2), _f32)],
    )
    return f(si2, msg2).reshape(_N, _FA)


def _sc_pool_body(x_hbm, ci2_hbm, seg_hbm, cnt_hbm, idxv, Xbuf, Obuf,
                  sacc, cacc):
    c = lax.axis_index("c")
    s = lax.axis_index("s")
    lo = c * _CHALF

    def orow(r, _):
        for q in range(4):
            Obuf[r, pl.ds(q * _L, _L)] = jnp.ones((_L,), _f32)
            Obuf[r, pl.ds((q + 4) * _L, _L)] = jnp.zeros((_L,), _f32)
        for q in range(8):
            Xbuf[r, pl.ds(q * _L, _L)] = jnp.zeros((_L,), _f32)
        return 0

    lax.fori_loop(0, _CH, orow, 0)

    @pl.when(s == 0)
    def _():
        pltpu.sync_copy(Xbuf, sacc.at[pl.ds(0, _CH)])
        pltpu.sync_copy(Xbuf.at[pl.ds(0, 8)], sacc.at[pl.ds(_CH, 8)])
        pltpu.sync_copy(Xbuf, cacc.at[pl.ds(0, _CH)])
        pltpu.sync_copy(Xbuf.at[pl.ds(0, 8)], cacc.at[pl.ds(_CH, 8)])

    plsc.subcore_barrier()

    def step(i, _):
        k = s + i * _NS

        @pl.when(k < _P_NCHUNK)
        def _():
            base = k * _CH
            pltpu.sync_copy(ci2_hbm.at[c, pl.ds(base, _CH)], idxv)
            pltpu.sync_copy(x_hbm.at[pl.ds(base, _CH)], Xbuf)
            pltpu.sync_copy(Xbuf, sacc.at[idxv], add=True)
            pltpu.sync_copy(Obuf, cacc.at[idxv], add=True)
        return 0

    lax.fori_loop(0, -(-_P_NCHUNK // _NS), step, 0)

    @pl.when(s == 0)
    def _():
        base = _P_NCHUNK * _CH
        pltpu.sync_copy(ci2_hbm.at[c, pl.ds(base, _CH)], idxv)
        pltpu.sync_copy(x_hbm.at[pl.ds(base, _P_TAIL)],
                        Xbuf.at[pl.ds(0, _P_TAIL)])
        pltpu.sync_copy(Xbuf, sacc.at[idxv], add=True)
        pltpu.sync_copy(Obuf, cacc.at[idxv], add=True)

    plsc.subcore_barrier()
    pltpu.sync_copy(sacc.at[pl.ds(s * 8, 8)],
                    seg_hbm.at[pl.ds(lo + s * 8, 8)])
    pltpu.sync_copy(cacc.at[pl.ds(s * 8, 8)],
                    cnt_hbm.at[pl.ds(lo + s * 8, 8)])


def _pool(xp, ci2):
    f = pl.kernel(
        _sc_pool_body,
        out_type=[jax.ShapeDtypeStruct((_C, _H2), _f32),
                  jax.ShapeDtypeStruct((_C, _H2), _f32)],
        mesh=plsc.VectorSubcoreMesh(core_axis_name="c", subcore_axis_name="s"),
        scratch_types=[pltpu.VMEM((_CH,), _i32),
                       pltpu.VMEM((_CH, _H2), _f32),
                       pltpu.VMEM((_CH, _H2), _f32),
                       pltpu.VMEM_SHARED((_CHALF + 8, _H2), _f32),
                       pltpu.VMEM_SHARED((_CHALF + 8, _H2), _f32)],
    )
    return f(xp, ci2)


def kernel(atom_fea, nbr_fea, self_fea_idx, nbr_fea_idx, crystal_atom_idx,
           W_emb, b_emb, conv_fc_W, conv_fc_b, bn1_g, bn1_b, bn2_g, bn2_b,
           W_c2f, b_c2f, W_out, b_out):
    self_fea_idx = self_fea_idx.astype(_i32)
    nbr_fea_idx = nbr_fea_idx.astype(_i32)
    crystal_atom_idx = crystal_atom_idx.astype(_i32)

    si_pad = jnp.pad(self_fea_idx, (0, 3200 * 256 - _E),
                     constant_values=_N).reshape(3200, 256)
    si2 = _prep(si_pad, _HALF, _GARB, 25, 128, 1).reshape(2, 3200 * 256)
    ci_pad = jnp.pad(crystal_atom_idx, (0, 200 * 256 - _N),
                     constant_values=_N).reshape(200, 256)
    ci2 = _prep(ci_pad, _CHALF, _CHALF, 25, 8, 0).reshape(2, 200 * 256)
    par = (self_fea_idx & 1).astype(_f32).reshape(_E, 1)

    x = _embed(atom_fea, W_emb, b_emb.reshape(1, _FA))
    for i in range(3):
        W = conv_fc_W[i]
        b2 = conv_fc_b[i].reshape(1, _H2)
        xs, xn = _atom_mm(x, W[0:_FA], W[_FA:2 * _FA])
        ep = _gather(self_fea_idx, nbr_fea_idx, xs, xn)
        wf = W[2 * _FA:]
        st = _stats(ep, nbr_fea, wf, b2)
        msg2 = _apply(ep, nbr_fea, wf, b2, st,
                      bn1_g[i].reshape(1, _H2), bn1_b[i].reshape(1, _H2), par)
        summed = _scatter(si2, msg2)
        ast = _astat(summed)
        x = _update(x, summed, ast,
                    bn2_g[i].reshape(1, _FA), bn2_b[i].reshape(1, _FA))
    seg, cnt = _pool(_padx(x), ci2)
    return _head(seg, cnt, W_c2f, b_c2f.reshape(1, 128),
                 W_out.reshape(1, 128), b_out.reshape(1, 1))
